# baseline (speedup 1.0000x reference)
import functools

import jax
import jax.numpy as jnp
from jax import lax
from jax.experimental import pallas as pl
from jax.experimental.pallas import tpu as pltpu
from jax.experimental.pallas import tpu_sc as plsc

N = 20000
E = 40000
ETOT = E + N
EPAD = 61440
NPAD = 20480
G = 512
IN_DIM = 64
HID = 128
OUT = 32
D48 = 48
NC, NS = 2, 16
NW = NC * NS

BE = 512
GBLK = EPAD // BE
BN = 800
NBLK = N // BN


def _make_sc_gather(n_chunks, d):
    rw = n_chunks * 128
    mesh = plsc.VectorSubcoreMesh(
        core_axis_name="c", subcore_axis_name="s", num_cores=NC,
        num_subcores=NS)

    @functools.partial(
        pl.kernel, mesh=mesh,
        compiler_params=pltpu.CompilerParams(use_tc_tiling_on_sc=False),
        out_type=jax.ShapeDtypeStruct((NW * rw, d), jnp.float32),
        scratch_types=[
            pltpu.VMEM((128,), jnp.int32),
            pltpu.VMEM((128, d), jnp.float32),
            pltpu.SemaphoreType.DMA,
        ],
    )
    def gather_k(table_hbm, idx_hbm, out_hbm, idxc, rbuf, sem):
        wid = lax.axis_index("s") * NC + lax.axis_index("c")

        def body(j, carry):
            pltpu.sync_copy(idx_hbm.at[wid, j], idxc)
            pltpu.async_copy(table_hbm.at[idxc], rbuf, sem).wait()
            pltpu.sync_copy(rbuf, out_hbm.at[pl.ds(wid * rw + j * 128, 128)])
            return carry

        lax.fori_loop(0, n_chunks, body, 0)

    return gather_k


def _make_sc_scatter(n_chunks, n_out):
    rw = n_chunks * 128
    stripe = n_out // NS
    mesh = plsc.VectorSubcoreMesh(
        core_axis_name="c", subcore_axis_name="s", num_cores=NC,
        num_subcores=NS)

    @functools.partial(
        pl.kernel, mesh=mesh,
        compiler_params=pltpu.CompilerParams(use_tc_tiling_on_sc=False),
        out_type=jax.ShapeDtypeStruct((NC, n_out, D48), jnp.float32),
        scratch_types=[
            pltpu.VMEM((128,), jnp.int32),
            pltpu.VMEM((128, D48), jnp.float32),
            pltpu.VMEM((stripe, D48), jnp.float32),
            pltpu.VMEM_SHARED((n_out, D48), jnp.float32),
        ],
    )
    def scatter_k(m_hbm, idx_hbm, zeros_hbm, out_hbm, idxc, mbuf, sbuf,
                  acc):
        cid = lax.axis_index("c")
        sid = lax.axis_index("s")
        wid = sid * NC + cid
        pltpu.sync_copy(zeros_hbm, sbuf)
        pltpu.sync_copy(sbuf, acc.at[pl.ds(sid * stripe, stripe)])
        plsc.subcore_barrier()

        def body(j, carry):
            pltpu.sync_copy(idx_hbm.at[wid, j], idxc)
            pltpu.sync_copy(m_hbm.at[pl.ds(wid * rw + j * 128, 128)], mbuf)
            pltpu.sync_copy(mbuf, acc.at[idxc], add=True)
            return carry

        lax.fori_loop(0, n_chunks, body, 0)
        plsc.subcore_barrier()
        pltpu.sync_copy(acc.at[pl.ds(sid * stripe, stripe)], sbuf)
        pltpu.sync_copy(sbuf, out_hbm.at[cid, pl.ds(sid * stripe, stripe)])

    return scatter_k


_sc_gather64 = _make_sc_gather(30, IN_DIM)
_sc_gather32 = _make_sc_gather(30, OUT)
_sc_scatter_conv = _make_sc_scatter(15, N)
_sc_scatter_pool = _make_sc_scatter(5, G)


def _bn_coefs(st_ref, gam_ref, bet_ref):
    mean = st_ref[0:1, :] * (1.0 / N)
    ex2 = st_ref[1:2, :] * (1.0 / N)
    var = ex2 - mean * mean
    a = gam_ref[...] * lax.rsqrt(var + 1e-5)
    b = bet_ref[...] - mean * a
    return a, b


def _msg_body(d_in, xs_ref, xd_ref, wf_ref, bf_ref, wo_ref, bo_ref, out_ref,
              *, normalize, st_ref=None, gam_ref=None, bet_ref=None):
    i = pl.program_id(0)
    xs = xs_ref[...]
    xd = xd_ref[...]
    if normalize:
        a, b = _bn_coefs(st_ref, gam_ref, bet_ref)
        xs = jnp.maximum(xs * a + b, 0.0)
        xd = jnp.maximum(xd * a + b, 0.0)
    freqs = jnp.dot(xd, wf_ref[...], preferred_element_type=jnp.float32)
    freqs = freqs + bf_ref[...]
    f3 = freqs.reshape(BE, OUT, d_in)
    proj = jnp.sum(f3 * xs[:, None, :], axis=2)
    s1 = jnp.sin(proj)
    c1 = jnp.cos(proj)
    s2 = 2.0 * s1 * c1
    c2 = c1 * c1 - s1 * s1
    s4 = 2.0 * s2 * c2
    c4 = c2 * c2 - s2 * s2
    emb = jnp.concatenate([s1, c1, s2, c2, s4, c4], axis=-1)
    m = jnp.dot(emb, wo_ref[...], preferred_element_type=jnp.float32)
    m = m + bo_ref[...]
    row = i * BE + lax.broadcasted_iota(jnp.int32, (BE, 1), 0)
    vf = (row < ETOT).astype(jnp.float32)
    out_ref[...] = jnp.concatenate(
        [m * vf, vf, jnp.zeros((BE, 15), jnp.float32)], axis=-1)


def _full(shape):
    return pl.BlockSpec(shape, lambda i: tuple(0 for _ in shape))


_msg1_call = pl.pallas_call(
    functools.partial(_msg_body, IN_DIM, normalize=False),
    grid=(GBLK,),
    in_specs=[
        pl.BlockSpec((BE, IN_DIM), lambda i: (i, 0)),
        pl.BlockSpec((BE, IN_DIM), lambda i: (i + GBLK, 0)),
        _full((IN_DIM, OUT * IN_DIM)),
        _full((1, OUT * IN_DIM)),
        _full((6 * OUT, OUT)),
        _full((1, OUT)),
    ],
    out_specs=pl.BlockSpec((BE, D48), lambda i: (i, 0)),
    out_shape=jax.ShapeDtypeStruct((EPAD, D48), jnp.float32),
)


def _msg2_body(xs_ref, xd_ref, wf_ref, bf_ref, wo_ref, bo_ref, st_ref,
               gam_ref, bet_ref, out_ref):
    _msg_body(OUT, xs_ref, xd_ref, wf_ref, bf_ref, wo_ref, bo_ref, out_ref,
              normalize=True, st_ref=st_ref, gam_ref=gam_ref, bet_ref=bet_ref)


_msg2_call = pl.pallas_call(
    _msg2_body,
    grid=(GBLK,),
    in_specs=[
        pl.BlockSpec((BE, OUT), lambda i: (i, 0)),
        pl.BlockSpec((BE, OUT), lambda i: (i + GBLK, 0)),
        _full((OUT, OUT * OUT)),
        _full((1, OUT * OUT)),
        _full((6 * OUT, OUT)),
        _full((1, OUT)),
        _full((8, OUT)),
        _full((1, OUT)),
        _full((1, OUT)),
    ],
    out_specs=pl.BlockSpec((BE, D48), lambda i: (i, 0)),
    out_shape=jax.ShapeDtypeStruct((EPAD, D48), jnp.float32),
)


def _upd_body(xin_ref, s_ref, wa_ref, ba_ref, wb_ref, bb_ref, u_ref, st_ref,
              *, normalize, st_in_ref=None, gam_ref=None, bet_ref=None):
    i = pl.program_id(0)
    xin = xin_ref[...]
    if normalize:
        a, b = _bn_coefs(st_in_ref, gam_ref, bet_ref)
        xin = jnp.maximum(xin * a + b, 0.0)
    s = s_ref[0, :, 0:OUT] + s_ref[1, :, 0:OUT]
    cnt = s_ref[0, :, OUT:OUT + 1] + s_ref[1, :, OUT:OUT + 1]
    agg = s / jnp.maximum(cnt, 1.0)
    cat = jnp.concatenate([xin, agg], axis=-1)
    hh = jnp.dot(cat, wa_ref[...], preferred_element_type=jnp.float32)
    hh = jnp.maximum(hh + ba_ref[...], 0.0)
    u = jnp.dot(hh, wb_ref[...], preferred_element_type=jnp.float32)
    u = u + bb_ref[...]
    u_ref[...] = u
    su = jnp.sum(u, axis=0, keepdims=True)
    sq = jnp.sum(u * u, axis=0, keepdims=True)

    @pl.when(i == 0)
    def _():
        st_ref[...] = jnp.zeros_like(st_ref)

    st_ref[0:1, :] = st_ref[0:1, :] + su
    st_ref[1:2, :] = st_ref[1:2, :] + sq


def _make_upd(d_in, normalize):
    if normalize:
        def body(xin_ref, st_in_ref, gam_ref, bet_ref, s_ref, wa_ref, ba_ref,
                 wb_ref, bb_ref, u_ref, st_ref):
            _upd_body(xin_ref, s_ref, wa_ref, ba_ref, wb_ref, bb_ref, u_ref,
                      st_ref, normalize=True, st_in_ref=st_in_ref,
                      gam_ref=gam_ref, bet_ref=bet_ref)
        extra = [_full((8, OUT)), _full((1, OUT)), _full((1, OUT))]
    else:
        def body(xin_ref, s_ref, wa_ref, ba_ref, wb_ref, bb_ref, u_ref,
                 st_ref):
            _upd_body(xin_ref, s_ref, wa_ref, ba_ref, wb_ref, bb_ref, u_ref,
                      st_ref, normalize=False)
        extra = []
    return pl.pallas_call(
        body,
        grid=(NBLK,),
        in_specs=[pl.BlockSpec((BN, d_in), lambda i: (i, 0))] + extra + [
            pl.BlockSpec((NC, BN, D48), lambda i: (0, i, 0)),
            _full((d_in + OUT, HID)),
            _full((1, HID)),
            _full((HID, OUT)),
            _full((1, OUT)),
        ],
        out_specs=[
            pl.BlockSpec((BN, OUT), lambda i: (i, 0)),
            pl.BlockSpec((8, OUT), lambda i: (0, 0)),
        ],
        out_shape=[
            jax.ShapeDtypeStruct((N, OUT), jnp.float32),
            jax.ShapeDtypeStruct((8, OUT), jnp.float32),
        ],
    )


_upd1_call = _make_upd(IN_DIM, normalize=False)
_upd2_call = _make_upd(OUT, normalize=True)


def _poolpre_body(u_ref, st_ref, gam_ref, bet_ref, wgt_ref, bg_ref, wn_ref,
                  bnn_ref, out_ref):
    a, b = _bn_coefs(st_ref, gam_ref, bet_ref)
    h = jnp.maximum(u_ref[...] * a + b, 0.0)
    gate = jnp.sum(h * wgt_ref[...], axis=-1, keepdims=True) + bg_ref[0, 0]
    gate = 1.0 / (1.0 + jnp.exp(-gate))
    e = jnp.exp(gate)
    hn = jnp.dot(h, wn_ref[...], preferred_element_type=jnp.float32)
    hn = jnp.maximum(hn + bnn_ref[...], 0.0)
    out_ref[...] = jnp.concatenate(
        [e * hn, e, jnp.zeros((BN, 15), jnp.float32)], axis=-1)


_poolpre_call = pl.pallas_call(
    _poolpre_body,
    grid=(NBLK,),
    in_specs=[
        pl.BlockSpec((BN, OUT), lambda i: (i, 0)),
        _full((8, OUT)),
        _full((1, OUT)),
        _full((1, OUT)),
        _full((1, OUT)),
        _full((1, 1)),
        _full((OUT, OUT)),
        _full((1, OUT)),
    ],
    out_specs=pl.BlockSpec((BN, D48), lambda i: (i, 0)),
    out_shape=jax.ShapeDtypeStruct((N, D48), jnp.float32),
)


def _final_body(p_ref, wc1_ref, bc1_ref, wc2_ref, bc2_ref, out_ref):
    p = p_ref[0] + p_ref[1]
    pooled = p[:, 0:OUT] / jnp.maximum(p[:, OUT:OUT + 1], 1e-16)
    hh = jnp.dot(pooled, wc1_ref[...], preferred_element_type=jnp.float32)
    hh = jnp.maximum(hh + bc1_ref[...], 0.0)
    o = jnp.dot(hh, wc2_ref[...], preferred_element_type=jnp.float32)
    out_ref[...] = o + bc2_ref[...]


_final_call = pl.pallas_call(
    _final_body,
    out_shape=jax.ShapeDtypeStruct((G, 10), jnp.float32),
)


def kernel(x, edge_index, batch, Wf1, bf1, Wo1, bo1, Wua1, bua1, Wub1, bub1,
           g1, be1, Wf2, bf2, Wo2, bo2, Wua2, bua2, Wub2, bub2, g2, be2, Wg,
           bg, Wn, bnn, Wc1, bc1, Wc2, bc2):
    loop = jnp.arange(N, dtype=jnp.int32)
    src = jnp.concatenate([edge_index[0], loop])
    dst = jnp.concatenate([edge_index[1], loop])
    pad_e = jnp.zeros((EPAD - ETOT,), jnp.int32)
    idx_g = jnp.concatenate([src, pad_e, dst, pad_e]).reshape(NW, 30, 128)
    idx_s = jnp.concatenate([dst, pad_e]).reshape(NW, 15, 128)
    idx_b = jnp.concatenate(
        [batch.astype(jnp.int32), jnp.zeros((NPAD - N,), jnp.int32)]
    ).reshape(NW, 5, 128)
    zconv = jnp.zeros((N // NS, D48), jnp.float32)
    zpool = jnp.zeros((G // NS, D48), jnp.float32)

    bf1r = bf1.reshape(1, -1)
    bo1r = bo1.reshape(1, -1)
    bua1r = bua1.reshape(1, -1)
    bub1r = bub1.reshape(1, -1)
    g1r = g1.reshape(1, -1)
    be1r = be1.reshape(1, -1)
    bf2r = bf2.reshape(1, -1)
    bo2r = bo2.reshape(1, -1)
    bua2r = bua2.reshape(1, -1)
    bub2r = bub2.reshape(1, -1)
    g2r = g2.reshape(1, -1)
    be2r = be2.reshape(1, -1)
    wgt = Wg.reshape(1, -1)
    bgr = bg.reshape(1, 1)
    bnnr = bnn.reshape(1, -1)
    bc1r = bc1.reshape(1, -1)
    bc2r = bc2.reshape(1, -1)

    gx = _sc_gather64(x, idx_g)
    m1 = _msg1_call(gx, gx, Wf1, bf1r, Wo1, bo1r)
    s1 = _sc_scatter_conv(m1, idx_s, zconv)
    u1, st1 = _upd1_call(x, s1, Wua1, bua1r, Wub1, bub1r)
    gh = _sc_gather32(u1, idx_g)
    m2 = _msg2_call(gh, gh, Wf2, bf2r, Wo2, bo2r, st1, g1r, be1r)
    s2 = _sc_scatter_conv(m2, idx_s, zconv)
    u2, st2 = _upd2_call(u1, st1, g1r, be1r, s2, Wua2, bua2r, Wub2, bub2r)
    pp = _poolpre_call(u2, st2, g2r, be2r, wgt, bgr, Wn, bnnr)
    ppad = jnp.pad(pp, ((0, NPAD - N), (0, 0)))
    pa = _sc_scatter_pool(ppad, idx_b, zpool)
    return _final_call(pa, Wc1, bc1r, Wc2, bc2r)
